# baseline (speedup 1.0000x reference)
import functools

import jax
import jax.numpy as jnp
from jax import lax
from jax.experimental import pallas as pl
from jax.experimental.pallas import tpu as pltpu
from jax.experimental.pallas import tpu_sc as plsc

CHAR_DIM = 32
WORD_DIM = 32
OUT_DIM = CHAR_DIM + WORD_DIM
BSLAB = 128
L_IDX = 8
L_OUT = 4
LANES = 16


@functools.lru_cache(maxsize=None)
def _make_sc_kernel(n_batch: int, seq_len: int):
    info = plsc.get_sparse_core_info()
    nw = info.num_cores * info.num_subcores
    assert n_batch == nw * BSLAB
    assert seq_len % L_IDX == 0 and L_IDX % L_OUT == 0
    n_super = seq_len // L_IDX
    nc = info.num_cores

    mesh = plsc.VectorSubcoreMesh(core_axis_name="c", subcore_axis_name="s")

    @functools.partial(
        pl.kernel,
        mesh=mesh,
        out_type=jax.ShapeDtypeStruct((seq_len, OUT_DIM, n_batch),
                                      jnp.float32),
        compiler_params=pltpu.CompilerParams(use_tc_tiling_on_sc=False,
                                             needs_layout_passes=False),
        scratch_types=[
            pltpu.VMEM((2, L_IDX, BSLAB), jnp.int32),
            pltpu.VMEM((2, 2, BSLAB, CHAR_DIM), jnp.float32),
            pltpu.VMEM((2, L_OUT, OUT_DIM, BSLAB), jnp.float32),
            pltpu.SemaphoreType.DMA((2,)),
            pltpu.SemaphoreType.DMA((2,)),
        ],
    )
    def k(voc_hbm, char_hbm, word_hbm, out_hbm, iv_v, gb_v, ob_v,
          sem_g, sem_w):
        wid = lax.axis_index("s") * nc + lax.axis_index("c")
        b0 = wid * BSLAB
        rowvs = [lax.iota(jnp.int32, LANES) + (LANES * kk)
                 for kk in range(BSLAB // LANES)]

        def gather_pair(l_local, pair):
            return [
                pltpu.make_async_copy(
                    char_hbm.at[iv_v.at[0, l_local]],
                    gb_v.at[pair, 0], sem_g.at[pair]),
                pltpu.make_async_copy(
                    word_hbm.at[iv_v.at[1, l_local]],
                    gb_v.at[pair, 1], sem_g.at[pair]),
            ]

        def transpose_into(pair, half, l_out):
            def dbody(d, carry):
                colv = jnp.full((LANES,), d, dtype=jnp.int32)
                for p in range(2):
                    for kk in range(BSLAB // LANES):
                        v = plsc.load_gather(gb_v.at[pair, p],
                                             [rowvs[kk], colv])
                        ob_v[half, l_out, p * CHAR_DIM + d,
                             pl.ds(LANES * kk, LANES)] = v
                return carry

            lax.fori_loop(0, CHAR_DIM, dbody, 0)

        def write_half(s, half):
            return pltpu.make_async_copy(
                ob_v.at[half],
                out_hbm.at[pl.ds(s * L_IDX + half * L_OUT, L_OUT), :,
                           pl.ds(b0, BSLAB)],
                sem_w.at[half])

        def super_body(s, carry):
            pltpu.sync_copy(
                voc_hbm.at[:, pl.ds(s * L_IDX, L_IDX), pl.ds(b0, BSLAB)],
                iv_v)
            for c in gather_pair(0, 0):
                c.start()
            for half in range(L_IDX // L_OUT):
                @pl.when(s > 0)
                def _drain_prev_write():
                    write_half(s - 1, half).wait()

                for j in range(L_OUT):
                    l_local = half * L_OUT + j
                    pair = l_local % 2
                    if l_local + 1 < L_IDX:
                        for c in gather_pair(l_local + 1, 1 - pair):
                            c.start()
                    for c in gather_pair(l_local, pair):
                        c.wait()
                    transpose_into(pair, half, j)
                write_half(s, half).start()
            return carry

        lax.fori_loop(0, n_super, super_body, 0)
        for half in range(L_IDX // L_OUT):
            write_half(n_super - 1, half).wait()

    return k


def kernel(voc, char_table, word_table):
    b, _, l = voc.shape
    if voc.dtype != jnp.int32:
        voc = voc.astype(jnp.int32)
    voc_t = jnp.transpose(voc, (1, 2, 0))
    out_t = _make_sc_kernel(b, l)(voc_t, char_table, word_table)
    return jnp.transpose(out_t, (2, 0, 1))
